# baseline (speedup 1.0000x reference)
import functools
import jax
import jax.numpy as jnp
from jax import lax
from jax.experimental import pallas as pl
from jax.experimental.pallas import tpu as pltpu
from jax.experimental.pallas import tpu_sc as plsc

_NUM_CORES = 2
_NUM_SUBCORES = 16
_NW = _NUM_CORES * _NUM_SUBCORES
_LANES = 16

_D = 64
_SCALE = 8.0
_IDX_ROW = 128
_ROWS_PER_CHUNK = 4
_CHUNK = _IDX_ROW * _ROWS_PER_CHUNK


def _make_gather(total_idx: int):
    assert total_idx % (_NW * _CHUNK) == 0
    per_worker = total_idx // _NW
    n_chunks = per_worker // _CHUNK
    idx_rows_per_worker = per_worker // _IDX_ROW

    mesh = plsc.VectorSubcoreMesh(core_axis_name="c", subcore_axis_name="s")

    @functools.partial(
        pl.kernel,
        out_type=jax.ShapeDtypeStruct((total_idx, _D), jnp.float32),
        mesh=mesh,
        scratch_types=[
            pltpu.VMEM((_ROWS_PER_CHUNK, _IDX_ROW), jnp.int32),
            pltpu.VMEM((_CHUNK, _D), jnp.float32),
            pltpu.SemaphoreType.DMA,
        ],
        compiler_params=pltpu.CompilerParams(use_tc_tiling_on_sc=False),
    )
    def gather_scale(idx_hbm, table_hbm, out_hbm, idx_v, rows_v, sem):
        wid = lax.axis_index("s") * _NUM_CORES + lax.axis_index("c")
        idx_row_base = wid * idx_rows_per_worker
        out_base = wid * per_worker

        def chunk_body(ci, _):
            pltpu.sync_copy(
                idx_hbm.at[pl.ds(idx_row_base + ci * _ROWS_PER_CHUNK,
                                 _ROWS_PER_CHUNK)],
                idx_v,
            )
            copies = []
            for r in range(_ROWS_PER_CHUNK):
                c = pltpu.make_async_copy(
                    table_hbm.at[idx_v.at[r]],
                    rows_v.at[pl.ds(r * _IDX_ROW, _IDX_ROW)],
                    sem,
                )
                c.start()
                copies.append(c)
            for c in copies:
                c.wait()

            def scale_body(i, _):
                for j in range(_D // _LANES):
                    sl = (i, pl.ds(j * _LANES, _LANES))
                    rows_v[sl] = rows_v[sl] * _SCALE
                return 0

            lax.fori_loop(0, _CHUNK, scale_body, 0, unroll=2)

            pltpu.sync_copy(
                rows_v,
                out_hbm.at[pl.ds(out_base + ci * _CHUNK, _CHUNK)],
            )
            return 0

        lax.fori_loop(0, n_chunks, chunk_body, 0)

    return gather_scale


def kernel(input_ids, table):
    b, t = input_ids.shape
    total = b * t
    idx2d = input_ids.reshape(total // _IDX_ROW, _IDX_ROW).astype(jnp.int32)
    out = _make_gather(total)(idx2d, table)
    return out.reshape(b, t, _D)
